# baseline (speedup 1.0000x reference)
import functools

import jax
import jax.numpy as jnp
from jax import lax
from jax.experimental import pallas as pl
from jax.experimental.pallas import tpu as pltpu
from jax.experimental.pallas import tpu_sc as plsc

_B, _S, _V, _D, _H, _T = 128, 200, 100000, 64, 64, 4
_ROWS = _B * _S
_NW = 32
_RPW = _ROWS // _NW
_NCH = 8
_CH = _RPW // _NCH

_CHUNK = 8
_NGRID = _S // _CHUNK
_VB = 2048
_NVB = (_V + _VB - 1) // _VB

_PREC = lax.Precision.HIGHEST


def _dot(a, b):
    return jnp.dot(a, b, preferred_element_type=jnp.float32, precision=_PREC)


def _gather_body(idx_hbm, table_hbm, out_hbm, idx_v, rows_v, sem):
    nc = plsc.get_sparse_core_info().num_cores
    wid = lax.axis_index("s") * nc + lax.axis_index("c")
    pltpu.sync_copy(idx_hbm.at[wid], idx_v)
    copies = [
        pltpu.async_copy(table_hbm.at[idx_v.at[j]],
                         rows_v.at[pl.ds(j * _CH, _CH)], sem)
        for j in range(_NCH)
    ]
    for c in copies:
        c.wait()
    pltpu.sync_copy(rows_v, out_hbm.at[pl.ds(wid * _RPW, _RPW)])


@jax.jit
def _sc_gather(idx, table):
    k = pl.kernel(
        _gather_body,
        mesh=plsc.VectorSubcoreMesh(core_axis_name="c", subcore_axis_name="s"),
        out_type=jax.ShapeDtypeStruct((_ROWS, _D), jnp.float32),
        scratch_types=[
            pltpu.VMEM((_NCH, _CH), jnp.int32),
            pltpu.VMEM((_RPW, _D), jnp.float32),
            pltpu.SemaphoreType.DMA,
        ],
        compiler_params=pltpu.CompilerParams(use_tc_tiling_on_sc=False),
    )
    return k(idx.reshape(_NW, _NCH, _CH), table)


def _scan_body(xe_ref, wx_ref, wh_ref, b_ref, wf_ref, bf_ref, dts_ref,
               hs_ref, h_ref):
    i = pl.program_id(0)

    @pl.when(i == 0)
    def _():
        h_ref[...] = jnp.zeros_like(h_ref)

    xe_c = xe_ref[...].reshape(_CHUNK * _B, _D)
    gx = _dot(xe_c, wx_ref[...]) + b_ref[...]

    h = h_ref[...]
    for t in range(_CHUNK):
        gx_t = gx[t * _B:(t + 1) * _B]
        gh = _dot(h, wh_ref[...])
        zr = jax.nn.sigmoid(gx_t[:, :2 * _H] + gh[:, :2 * _H])
        z = zr[:, :_H]
        r = zr[:, _H:]
        n = jnp.tanh(gx_t[:, 2 * _H:] + r * gh[:, 2 * _H:])
        h = (1.0 - z) * h + z * n
    h_ref[...] = h

    @pl.when(i == _NGRID - 1)
    def _():
        def f(hh):
            return jnp.tanh(_dot(hh, wf_ref[...]) + bf_ref[...])

        hs_ref[0:_B, :] = h
        hc = h
        for s in range(_T - 1):
            dt = dts_ref[s]
            k1 = f(hc)
            k2 = f(hc + 0.5 * dt * k1)
            k3 = f(hc + 0.5 * dt * k2)
            k4 = f(hc + dt * k3)
            hc = hc + (dt / 6.0) * (k1 + 2.0 * k2 + 2.0 * k3 + k4)
            hs_ref[(s + 1) * _B:(s + 2) * _B, :] = hc


@jax.jit
def _scan_call(xe, Wx, Wh, b, Wf, bf, dts):
    return pl.pallas_call(
        _scan_body,
        grid=(_NGRID,),
        in_specs=[
            pl.BlockSpec((_CHUNK, _B, _D), lambda i: (i, 0, 0)),
            pl.BlockSpec((_D, 3 * _H), lambda i: (0, 0)),
            pl.BlockSpec((_H, 3 * _H), lambda i: (0, 0)),
            pl.BlockSpec((1, 3 * _H), lambda i: (0, 0)),
            pl.BlockSpec((_H, _H), lambda i: (0, 0)),
            pl.BlockSpec((1, _H), lambda i: (0, 0)),
            pl.BlockSpec(memory_space=pltpu.SMEM),
        ],
        out_specs=pl.BlockSpec((_T * _B, _H), lambda i: (0, 0)),
        out_shape=jax.ShapeDtypeStruct((_T * _B, _H), jnp.float32),
        scratch_shapes=[pltpu.VMEM((_B, _H), jnp.float32)],
    )(xe, Wx, Wh, b, Wf, bf, dts)


def _proj_body(hs_ref, wout_ref, bout_ref, out_ref):
    out_ref[...] = _dot(hs_ref[...], wout_ref[...]) + bout_ref[...]


@jax.jit
def _proj_call(hs, Wout, bout):
    return pl.pallas_call(
        _proj_body,
        grid=(_NVB,),
        in_specs=[
            pl.BlockSpec((_T * _B, _H), lambda j: (0, 0)),
            pl.BlockSpec((_H, _VB), lambda j: (0, j)),
            pl.BlockSpec((1, _VB), lambda j: (0, j)),
        ],
        out_specs=pl.BlockSpec((_T * _B, _VB), lambda j: (0, j)),
        out_shape=jax.ShapeDtypeStruct((_T * _B, _V), jnp.float32),
    )(hs, Wout, bout)


def kernel(x, t_span, emb_table, Wx, Wh, b, Wf, bf, Wout, bout):
    idx = x.T.reshape(-1).astype(jnp.int32)
    xe = _sc_gather(idx, emb_table)
    dts = t_span[1:] - t_span[:-1]
    hs = _scan_call(xe.reshape(_S, _B, _D), Wx, Wh, b.reshape(1, -1),
                    Wf, bf.reshape(1, -1), dts)
    out = _proj_call(hs, Wout, bout.reshape(1, -1))
    return out.reshape(_T, _B, _V)
